# baseline (speedup 1.0000x reference)
import dataclasses
import functools

import jax
import jax.numpy as jnp
from jax import lax
from jax.experimental import pallas as pl
from jax.experimental.pallas import tpu as pltpu
from jax.experimental.pallas import tpu_sc as plsc

N_NODES = 10000
N_EDGES = 320000
N_GRAPHS = 64
N_CLASSES = 10

_NC = 2
_NS = 16
_NW = _NC * _NS
_K = 128
_CHUNKS_PER_W = 80
_HALF = _CHUNKS_PER_W // 2
_N_CHUNKS = _CHUNKS_PER_W * _NW
_E_PAD = _N_CHUNKS * _K

_ROWS_PER_TILE = 624
_ROWS_REM = N_NODES - _NS * _ROWS_PER_TILE

_HIGH = lax.Precision.HIGHEST


def _dot(a, b):
    return lax.dot_general(a, b, (((1,), (0,)), ((), ())),
                           preferred_element_type=jnp.float32,
                           precision=_HIGH)


def _mm(x, w):
    def body(x_ref, w_ref, o_ref):
        o_ref[...] = _dot(x_ref[...], w_ref[...])
    return pl.pallas_call(
        body,
        out_shape=jax.ShapeDtypeStruct((x.shape[0], w.shape[1]), jnp.float32),
    )(x, w)


def _fuse(acc, b, w):
    def body(a_ref, b_ref, w_ref, o_ref):
        h = jnp.maximum(a_ref[0] + a_ref[1] + b_ref[...], 0.0)
        o_ref[...] = _dot(h, w_ref[...])
    return pl.pallas_call(
        body,
        out_shape=jax.ShapeDtypeStruct((acc.shape[1], w.shape[1]), jnp.float32),
    )(acc, b.reshape(1, -1), w)


def _final(acc, b, batch2d, wlin, blin):
    def body(a_ref, b_ref, bt_ref, wl_ref, bl_ref, o_ref):
        out3 = a_ref[0] + a_ref[1] + b_ref[...]
        gi = lax.broadcasted_iota(jnp.int32, (N_NODES, N_GRAPHS), 1)
        onehot = (bt_ref[...] == gi).astype(jnp.float32)
        sums = lax.dot_general(onehot, out3, (((0,), (0,)), ((), ())),
                               preferred_element_type=jnp.float32,
                               precision=_HIGH)
        ones = jnp.ones((N_NODES, 1), jnp.float32)
        counts = lax.dot_general(onehot, ones, (((0,), (0,)), ((), ())),
                                 preferred_element_type=jnp.float32,
                                 precision=_HIGH)
        pooled = sums / jnp.maximum(counts, 1.0)
        o_ref[...] = _dot(pooled, wl_ref[...]) + bl_ref[...]
    return pl.pallas_call(
        body,
        out_shape=jax.ShapeDtypeStruct((N_GRAPHS, N_CLASSES), jnp.float32),
    )(acc, b.reshape(1, -1), batch2d, wlin, blin.reshape(1, -1))


def _make_scatter(d):
    mesh = plsc.VectorSubcoreMesh(core_axis_name="c", subcore_axis_name="s")
    cp = pltpu.CompilerParams()
    if "needs_layout_passes" in pltpu.CompilerParams.__dataclass_fields__:
        cp = dataclasses.replace(cp, needs_layout_passes=False)
    if d < 128 and "use_tc_tiling_on_sc" in pltpu.CompilerParams.__dataclass_fields__:
        cp = dataclasses.replace(cp, use_tc_tiling_on_sc=False)

    nj = d // 16

    @functools.partial(
        pl.kernel,
        compiler_params=cp,
        out_type=jax.ShapeDtypeStruct((_NC, N_NODES, d), jnp.float32),
        mesh=mesh,
        scratch_types=(
            [pltpu.VMEM((_K, d), jnp.float32) for _ in range(2)]
            + [pltpu.VMEM((_HALF, _K), jnp.int32),
               pltpu.VMEM((_HALF, _K), jnp.float32)]
            + [pltpu.VMEM((_K,), jnp.int32) for _ in range(2)]
            + [pltpu.VMEM_SHARED((N_NODES, d), jnp.float32)]
            + [pltpu.SemaphoreType.DMA for _ in range(4)]
        ),
    )
    def sc_kernel(src_hbm, dst_hbm, w_hbm, h_hbm, z_hbm, out_hbm, *scr):
        rows = scr[0:2]
        sb, wb = scr[2], scr[3]
        dv = scr[4:6]
        acc = scr[6]
        gsem = scr[7:9]
        dsem = scr[9:11]

        c = lax.axis_index("c")
        s = lax.axis_index("s")
        wid = s * _NC + c
        r0 = s * _ROWS_PER_TILE
        ch0 = wid * _CHUNKS_PER_W

        def start_gather(j, b):
            pltpu.async_copy(h_hbm.at[sb.at[j]], rows[b], gsem[b])

        def wait_gather(j, b):
            pltpu.make_async_copy(h_hbm.at[sb.at[j]], rows[b], gsem[b]).wait()

        def start_dst(g, b):
            pltpu.async_copy(dst_hbm.at[pl.ds(g * _K, _K)], dv[b], dsem[b])

        def wait_dst(g, b):
            pltpu.make_async_copy(dst_hbm.at[pl.ds(g * _K, _K)],
                                  dv[b], dsem[b]).wait()

        def sync_scatter(b):
            pltpu.sync_copy(rows[b], acc.at[dv[b]], add=True)

        def multiply(j, b):
            @pl.loop(0, _K, step=4)
            def _(k0):
                for kk in range(4):
                    k = k0 + kk
                    wv = plsc.load_gather(
                        wb, [jnp.full((16,), 0, jnp.int32) + j,
                             jnp.full((16,), 0, jnp.int32) + k])
                    for jj in range(nj):
                        sl = (k, pl.ds(jj * 16, 16))
                        rows[b][sl] = rows[b][sl] * wv

        pltpu.sync_copy(z_hbm.at[pl.ds(r0, _ROWS_PER_TILE)],
                        acc.at[pl.ds(r0, _ROWS_PER_TILE)])

        @pl.when(s == _NS - 1)
        def _():
            pltpu.sync_copy(z_hbm.at[pl.ds(_NS * _ROWS_PER_TILE, _ROWS_REM)],
                            acc.at[pl.ds(_NS * _ROWS_PER_TILE, _ROWS_REM)])

        plsc.subcore_barrier()

        for half in range(2):
            hc0 = ch0 + half * _HALF
            pltpu.sync_copy(src_hbm.at[pl.ds(hc0, _HALF)], sb)
            pltpu.sync_copy(w_hbm.at[pl.ds(hc0, _HALF)], wb)

            start_gather(0, 0)
            start_dst(hc0, 0)

            @pl.loop(0, _HALF, step=2)
            def _(j0):
                for b in range(2):
                    j = j0 + b

                    @pl.when(j + 1 < _HALF)
                    def _():
                        start_gather(j + 1, 1 - b)
                        start_dst(hc0 + j + 1, 1 - b)

                    wait_gather(j, b)
                    multiply(j, b)
                    wait_dst(hc0 + j, b)
                    sync_scatter(b)

        plsc.subcore_barrier()
        pltpu.sync_copy(acc.at[pl.ds(r0, _ROWS_PER_TILE)],
                        out_hbm.at[c, pl.ds(r0, _ROWS_PER_TILE)])

        @pl.when(s == _NS - 1)
        def _():
            pltpu.sync_copy(acc.at[pl.ds(_NS * _ROWS_PER_TILE, _ROWS_REM)],
                            out_hbm.at[c, pl.ds(_NS * _ROWS_PER_TILE, _ROWS_REM)])

    return sc_kernel


_scatter128 = _make_scatter(128)
_scatter64 = _make_scatter(64)


@jax.jit
def kernel(x, edge_index, batch, edge_weights, W1, b1, W2, b2, W3, b3,
           Wlin, blin):
    src = edge_index[0].astype(jnp.int32)
    dst = edge_index[1].astype(jnp.int32)
    pad = _E_PAD - N_EDGES
    src_p = jnp.concatenate([src, jnp.zeros((pad,), jnp.int32)]
                            ).reshape(_N_CHUNKS, _K)
    dst_p = jnp.concatenate([dst, jnp.zeros((pad,), jnp.int32)])
    w_p = jnp.concatenate([edge_weights.astype(jnp.float32),
                           jnp.zeros((pad,), jnp.float32)]
                          ).reshape(_N_CHUNKS, _K)

    z128 = jnp.zeros((N_NODES, 128), jnp.float32)
    z64 = jnp.zeros((N_NODES, 64), jnp.float32)
    batch2d = batch.astype(jnp.int32).reshape(N_NODES, 1)

    h1 = _mm(x, W1)
    a1 = _scatter128(src_p, dst_p, w_p, h1, z128)
    h2 = _fuse(a1, b1, W2)
    a2 = _scatter128(src_p, dst_p, w_p, h2, z128)
    h3 = _fuse(a2, b2, W3)
    a3 = _scatter64(src_p, dst_p, w_p, h3, z64)
    return _final(a3, b3, batch2d, Wlin, blin)


# speedup vs baseline: 1.0390x; 1.0390x over previous
import dataclasses
import functools

import jax
import jax.numpy as jnp
from jax import lax
from jax.experimental import pallas as pl
from jax.experimental.pallas import tpu as pltpu
from jax.experimental.pallas import tpu_sc as plsc

N_NODES = 10000
N_EDGES = 320000
N_GRAPHS = 64
N_CLASSES = 10

_NC = 2
_NS = 16
_NW = _NC * _NS
_K = 128
_CHUNKS_PER_W = 80
_HALF = _CHUNKS_PER_W // 2
_N_CHUNKS = _CHUNKS_PER_W * _NW
_E_PAD = _N_CHUNKS * _K

_ROWS_PER_TILE = 624
_ROWS_REM = N_NODES - _NS * _ROWS_PER_TILE

_HIGH = lax.Precision.HIGHEST


def _dot(a, b):
    return lax.dot_general(a, b, (((1,), (0,)), ((), ())),
                           preferred_element_type=jnp.float32,
                           precision=_HIGH)


def _mm(x, w):
    def body(x_ref, w_ref, o_ref):
        o_ref[...] = _dot(x_ref[...], w_ref[...])
    return pl.pallas_call(
        body,
        out_shape=jax.ShapeDtypeStruct((x.shape[0], w.shape[1]), jnp.float32),
    )(x, w)


def _fuse(acc, b, w):
    def body(a_ref, b_ref, w_ref, o_ref):
        h = jnp.maximum(a_ref[0] + a_ref[1] + b_ref[...], 0.0)
        o_ref[...] = _dot(h, w_ref[...])
    return pl.pallas_call(
        body,
        out_shape=jax.ShapeDtypeStruct((acc.shape[1], w.shape[1]), jnp.float32),
    )(acc, b.reshape(1, -1), w)


def _final(acc, b, batch2d, wlin, blin):
    def body(a_ref, b_ref, bt_ref, wl_ref, bl_ref, o_ref):
        out3 = a_ref[0] + a_ref[1] + b_ref[...]
        gi = lax.broadcasted_iota(jnp.int32, (N_NODES, N_GRAPHS), 1)
        onehot = (bt_ref[...] == gi).astype(jnp.float32)
        sums = lax.dot_general(onehot, out3, (((0,), (0,)), ((), ())),
                               preferred_element_type=jnp.float32,
                               precision=_HIGH)
        ones = jnp.ones((N_NODES, 1), jnp.float32)
        counts = lax.dot_general(onehot, ones, (((0,), (0,)), ((), ())),
                                 preferred_element_type=jnp.float32,
                                 precision=_HIGH)
        pooled = sums / jnp.maximum(counts, 1.0)
        o_ref[...] = _dot(pooled, wl_ref[...]) + bl_ref[...]
    return pl.pallas_call(
        body,
        out_shape=jax.ShapeDtypeStruct((N_GRAPHS, N_CLASSES), jnp.float32),
    )(acc, b.reshape(1, -1), batch2d, wlin, blin.reshape(1, -1))


def _make_scatter(d):
    mesh = plsc.VectorSubcoreMesh(core_axis_name="c", subcore_axis_name="s")
    cp = pltpu.CompilerParams()
    if "needs_layout_passes" in pltpu.CompilerParams.__dataclass_fields__:
        cp = dataclasses.replace(cp, needs_layout_passes=False)
    if d < 128 and "use_tc_tiling_on_sc" in pltpu.CompilerParams.__dataclass_fields__:
        cp = dataclasses.replace(cp, use_tc_tiling_on_sc=False)

    nj = d // 16

    @functools.partial(
        pl.kernel,
        compiler_params=cp,
        out_type=jax.ShapeDtypeStruct((_NC, N_NODES, d), jnp.float32),
        mesh=mesh,
        scratch_types=(
            [pltpu.VMEM((_K, d), jnp.float32) for _ in range(2)]
            + [pltpu.VMEM((_K,), jnp.int32) for _ in range(4)]
            + [pltpu.VMEM((_K,), jnp.int32) for _ in range(4)]
            + [pltpu.VMEM((_K,), jnp.float32) for _ in range(4)]
            + [pltpu.VMEM_SHARED((N_NODES, d), jnp.float32)]
            + [pltpu.SemaphoreType.DMA for _ in range(6)]
        ),
    )
    def sc_kernel(src_hbm, dst_hbm, w_hbm, h_hbm, z_hbm, out_hbm, *scr):
        rows = scr[0:2]
        sv = scr[2:6]
        dv = scr[6:10]
        wv = scr[10:14]
        acc = scr[14]
        gsem = scr[15:17]
        esem = scr[17:21]

        c = lax.axis_index("c")
        s = lax.axis_index("s")
        wid = s * _NC + c
        r0 = s * _ROWS_PER_TILE

        def start_edata(i, e):
            g = i * _NW + wid
            pltpu.async_copy(src_hbm.at[pl.ds(g * _K, _K)], sv[e], esem[e])
            pltpu.async_copy(dst_hbm.at[pl.ds(g * _K, _K)], dv[e], esem[e])
            pltpu.async_copy(w_hbm.at[pl.ds(g * _K, _K)], wv[e], esem[e])

        def wait_edata(i, e):
            g = i * _NW + wid
            pltpu.make_async_copy(src_hbm.at[pl.ds(g * _K, _K)],
                                  sv[e], esem[e]).wait()
            pltpu.make_async_copy(dst_hbm.at[pl.ds(g * _K, _K)],
                                  dv[e], esem[e]).wait()
            pltpu.make_async_copy(w_hbm.at[pl.ds(g * _K, _K)],
                                  wv[e], esem[e]).wait()

        def start_gather(e, b):
            pltpu.async_copy(h_hbm.at[sv[e]], rows[b], gsem[b])

        def wait_gather(e, b):
            pltpu.make_async_copy(h_hbm.at[sv[e]], rows[b], gsem[b]).wait()

        def sync_scatter(e, b):
            pltpu.sync_copy(rows[b], acc.at[dv[e]], add=True)

        def multiply(e, b):
            @pl.loop(0, _K, step=4)
            def _(k0):
                for kk in range(4):
                    k = k0 + kk
                    wvec = plsc.load_gather(
                        wv[e], [jnp.full((16,), 0, jnp.int32) + k])
                    for jj in range(nj):
                        sl = (k, pl.ds(jj * 16, 16))
                        rows[b][sl] = rows[b][sl] * wvec

        pltpu.sync_copy(z_hbm.at[pl.ds(r0, _ROWS_PER_TILE)],
                        acc.at[pl.ds(r0, _ROWS_PER_TILE)])

        @pl.when(s == _NS - 1)
        def _():
            pltpu.sync_copy(z_hbm.at[pl.ds(_NS * _ROWS_PER_TILE, _ROWS_REM)],
                            acc.at[pl.ds(_NS * _ROWS_PER_TILE, _ROWS_REM)])

        plsc.subcore_barrier()

        start_edata(0, 0)
        start_edata(1, 1)
        wait_edata(0, 0)
        start_gather(0, 0)

        @pl.loop(0, _CHUNKS_PER_W, step=4)
        def _(j0):
            for b4 in range(4):
                j = j0 + b4
                e = b4
                b = b4 % 2

                @pl.when(j + 2 < _CHUNKS_PER_W)
                def _():
                    start_edata(j + 2, (b4 + 2) % 4)

                @pl.when(j + 1 < _CHUNKS_PER_W)
                def _():
                    wait_edata(j + 1, (b4 + 1) % 4)
                    start_gather((b4 + 1) % 4, 1 - b)

                wait_gather(e, b)
                multiply(e, b)
                sync_scatter(e, b)

        plsc.subcore_barrier()
        pltpu.sync_copy(acc.at[pl.ds(r0, _ROWS_PER_TILE)],
                        out_hbm.at[c, pl.ds(r0, _ROWS_PER_TILE)])

        @pl.when(s == _NS - 1)
        def _():
            pltpu.sync_copy(acc.at[pl.ds(_NS * _ROWS_PER_TILE, _ROWS_REM)],
                            out_hbm.at[c, pl.ds(_NS * _ROWS_PER_TILE, _ROWS_REM)])

    return sc_kernel


_scatter128 = _make_scatter(128)
_scatter64 = _make_scatter(64)


@jax.jit
def kernel(x, edge_index, batch, edge_weights, W1, b1, W2, b2, W3, b3,
           Wlin, blin):
    src = edge_index[0].astype(jnp.int32)
    dst = edge_index[1].astype(jnp.int32)
    pad = _E_PAD - N_EDGES
    src_p = jnp.concatenate([src, jnp.zeros((pad,), jnp.int32)])
    dst_p = jnp.concatenate([dst, jnp.zeros((pad,), jnp.int32)])
    w_p = jnp.concatenate([edge_weights.astype(jnp.float32),
                           jnp.zeros((pad,), jnp.float32)])

    z128 = jnp.zeros((N_NODES, 128), jnp.float32)
    z64 = jnp.zeros((N_NODES, 64), jnp.float32)
    batch2d = batch.astype(jnp.int32).reshape(N_NODES, 1)

    h1 = _mm(x, W1)
    a1 = _scatter128(src_p, dst_p, w_p, h1, z128)
    h2 = _fuse(a1, b1, W2)
    a2 = _scatter128(src_p, dst_p, w_p, h2, z128)
    h3 = _fuse(a2, b2, W3)
    a3 = _scatter64(src_p, dst_p, w_p, h3, z64)
    return _final(a3, b3, batch2d, Wlin, blin)


# speedup vs baseline: 1.1465x; 1.1035x over previous
import dataclasses
import functools

import jax
import jax.numpy as jnp
from jax import lax
from jax.experimental import pallas as pl
from jax.experimental.pallas import tpu as pltpu
from jax.experimental.pallas import tpu_sc as plsc

N_NODES = 10000
N_EDGES = 320000
N_GRAPHS = 64
N_CLASSES = 10

_NC = 2
_NS = 16
_NW = _NC * _NS
_K = 128
_N_CHUNKS = N_EDGES // _K
_CHUNKS_PER_W = (_N_CHUNKS + _NW - 1) // _NW
_ROWS_PER_TILE = 624
_ROWS_REM = N_NODES - _NS * _ROWS_PER_TILE

_HIGH = lax.Precision.HIGHEST


def _dot(a, b):
    return lax.dot_general(a, b, (((1,), (0,)), ((), ())),
                           preferred_element_type=jnp.float32,
                           precision=_HIGH)


def _mm(x, w):
    def body(x_ref, w_ref, o_ref):
        o_ref[...] = _dot(x_ref[...], w_ref[...])
    return pl.pallas_call(
        body,
        out_shape=jax.ShapeDtypeStruct((x.shape[0], w.shape[1]), jnp.float32),
    )(x, w)


def _fuse(acc, b, w):
    def body(a_ref, b_ref, w_ref, o_ref):
        h = jnp.maximum(a_ref[0] + a_ref[1] + b_ref[...], 0.0)
        o_ref[...] = _dot(h, w_ref[...])
    return pl.pallas_call(
        body,
        out_shape=jax.ShapeDtypeStruct((acc.shape[1], w.shape[1]), jnp.float32),
    )(acc, b.reshape(1, -1), w)


def _final(acc, b, batch2d, wlin, blin):
    def body(a_ref, b_ref, bt_ref, wl_ref, bl_ref, o_ref):
        out3 = a_ref[0] + a_ref[1] + b_ref[...]
        gi = lax.broadcasted_iota(jnp.int32, (N_NODES, N_GRAPHS), 1)
        onehot = (bt_ref[...] == gi).astype(jnp.float32)
        sums = lax.dot_general(onehot, out3, (((0,), (0,)), ((), ())),
                               preferred_element_type=jnp.float32,
                               precision=_HIGH)
        ones = jnp.ones((N_NODES, 1), jnp.float32)
        counts = lax.dot_general(onehot, ones, (((0,), (0,)), ((), ())),
                                 preferred_element_type=jnp.float32,
                                 precision=_HIGH)
        pooled = sums / jnp.maximum(counts, 1.0)
        o_ref[...] = _dot(pooled, wl_ref[...]) + bl_ref[...]
    return pl.pallas_call(
        body,
        out_shape=jax.ShapeDtypeStruct((N_GRAPHS, N_CLASSES), jnp.float32),
    )(acc, b.reshape(1, -1), batch2d, wlin, blin.reshape(1, -1))


def _make_scatter(d):
    mesh = plsc.VectorSubcoreMesh(core_axis_name="c", subcore_axis_name="s")
    cp = pltpu.CompilerParams()
    if "needs_layout_passes" in pltpu.CompilerParams.__dataclass_fields__:
        cp = dataclasses.replace(cp, needs_layout_passes=False)
    if d < 128 and "use_tc_tiling_on_sc" in pltpu.CompilerParams.__dataclass_fields__:
        cp = dataclasses.replace(cp, use_tc_tiling_on_sc=False)

    @functools.partial(
        pl.kernel,
        compiler_params=cp,
        out_type=jax.ShapeDtypeStruct((_NC, N_NODES, d), jnp.float32),
        mesh=mesh,
        scratch_types=[
            pltpu.VMEM((_K,), jnp.int32),
            pltpu.VMEM((_K,), jnp.int32),
            pltpu.VMEM((_K,), jnp.float32),
            pltpu.VMEM((_K, d), jnp.float32),
            pltpu.VMEM_SHARED((N_NODES, d), jnp.float32),
            pltpu.SemaphoreType.DMA,
        ],
    )
    def sc_kernel(h_hbm, src_hbm, dst_hbm, w_hbm, z_hbm, out_hbm,
                  srcv, dstv, wv, rows, acc, sem):
        c = lax.axis_index("c")
        s = lax.axis_index("s")
        wid = s * _NC + c
        r0 = s * _ROWS_PER_TILE

        pltpu.sync_copy(z_hbm.at[pl.ds(r0, _ROWS_PER_TILE)],
                        acc.at[pl.ds(r0, _ROWS_PER_TILE)])

        @pl.when(s == _NS - 1)
        def _():
            pltpu.sync_copy(z_hbm.at[pl.ds(_NS * _ROWS_PER_TILE, _ROWS_REM)],
                            acc.at[pl.ds(_NS * _ROWS_PER_TILE, _ROWS_REM)])

        plsc.subcore_barrier()

        @pl.loop(0, _CHUNKS_PER_W)
        def _(i):
            ci = i * _NW + wid

            @pl.when(ci < _N_CHUNKS)
            def _():
                e0 = ci * _K
                pltpu.sync_copy(src_hbm.at[pl.ds(e0, _K)], srcv)
                pltpu.sync_copy(dst_hbm.at[pl.ds(e0, _K)], dstv)
                pltpu.sync_copy(w_hbm.at[pl.ds(e0, _K)], wv)
                pltpu.async_copy(h_hbm.at[srcv], rows, sem).wait()

                @pl.loop(0, _K, step=4)
                def _(k0):
                    for kk in range(4):
                        k = k0 + kk
                        wb = plsc.load_gather(wv, [jnp.full((16,), 0, jnp.int32) + k])
                        for j in range(d // 16):
                            sl = (k, pl.ds(j * 16, 16))
                            rows[sl] = rows[sl] * wb

                pltpu.sync_copy(rows, acc.at[dstv], add=True)

        plsc.subcore_barrier()
        pltpu.sync_copy(acc.at[pl.ds(r0, _ROWS_PER_TILE)],
                        out_hbm.at[c, pl.ds(r0, _ROWS_PER_TILE)])

        @pl.when(s == _NS - 1)
        def _():
            pltpu.sync_copy(acc.at[pl.ds(_NS * _ROWS_PER_TILE, _ROWS_REM)],
                            out_hbm.at[c, pl.ds(_NS * _ROWS_PER_TILE, _ROWS_REM)])

    return sc_kernel


_scatter128 = _make_scatter(128)
_scatter64 = _make_scatter(64)


@jax.jit
def kernel(x, edge_index, batch, edge_weights, W1, b1, W2, b2, W3, b3,
           Wlin, blin):
    src = edge_index[0].astype(jnp.int32)
    dst = edge_index[1].astype(jnp.int32)
    z128 = jnp.zeros((N_NODES, 128), jnp.float32)
    z64 = jnp.zeros((N_NODES, 64), jnp.float32)
    batch2d = batch.astype(jnp.int32).reshape(N_NODES, 1)

    h1 = _mm(x, W1)
    a1 = _scatter128(h1, src, dst, edge_weights, z128)
    h2 = _fuse(a1, b1, W2)
    a2 = _scatter128(h2, src, dst, edge_weights, z128)
    h3 = _fuse(a2, b2, W3)
    a3 = _scatter64(h3, src, dst, edge_weights, z64)
    return _final(a3, b3, batch2d, Wlin, blin)


# speedup vs baseline: 1.4169x; 1.2359x over previous
import dataclasses
import functools

import jax
import jax.numpy as jnp
from jax import lax
from jax.experimental import pallas as pl
from jax.experimental.pallas import tpu as pltpu
from jax.experimental.pallas import tpu_sc as plsc

N_NODES = 10000
N_EDGES = 320000
N_GRAPHS = 64
N_CLASSES = 10

_NC = 2
_NS = 16
_NW = _NC * _NS
_K = 128
_N_CHUNKS = N_EDGES // _K
_CHUNKS_PER_W = (_N_CHUNKS + _NW - 1) // _NW
_ROWS_PER_TILE = 624
_ROWS_REM = N_NODES - _NS * _ROWS_PER_TILE

_HIGH = lax.Precision.HIGHEST


def _dot(a, b):
    return lax.dot_general(a, b, (((1,), (0,)), ((), ())),
                           preferred_element_type=jnp.float32,
                           precision=_HIGH)


def _mm(x, w):
    def body(x_ref, w_ref, o_ref):
        o_ref[...] = _dot(x_ref[...], w_ref[...])
    return pl.pallas_call(
        body,
        out_shape=jax.ShapeDtypeStruct((x.shape[0], w.shape[1]), jnp.float32),
    )(x, w)


def _fuse(acc, b, w):
    def body(a_ref, b_ref, w_ref, o_ref):
        h = jnp.maximum(a_ref[0] + a_ref[1] + b_ref[...], 0.0)
        o_ref[...] = _dot(h, w_ref[...])
    return pl.pallas_call(
        body,
        out_shape=jax.ShapeDtypeStruct((acc.shape[1], w.shape[1]), jnp.float32),
    )(acc, b.reshape(1, -1), w)


def _final(acc, b, batch2d, wlin, blin):
    def body(a_ref, b_ref, bt_ref, wl_ref, bl_ref, o_ref):
        out3 = a_ref[0] + a_ref[1] + b_ref[...]
        gi = lax.broadcasted_iota(jnp.int32, (N_NODES, N_GRAPHS), 1)
        onehot = (bt_ref[...] == gi).astype(jnp.float32)
        sums = lax.dot_general(onehot, out3, (((0,), (0,)), ((), ())),
                               preferred_element_type=jnp.float32,
                               precision=_HIGH)
        ones = jnp.ones((N_NODES, 1), jnp.float32)
        counts = lax.dot_general(onehot, ones, (((0,), (0,)), ((), ())),
                                 preferred_element_type=jnp.float32,
                                 precision=_HIGH)
        pooled = sums / jnp.maximum(counts, 1.0)
        o_ref[...] = _dot(pooled, wl_ref[...]) + bl_ref[...]
    return pl.pallas_call(
        body,
        out_shape=jax.ShapeDtypeStruct((N_GRAPHS, N_CLASSES), jnp.float32),
    )(acc, b.reshape(1, -1), batch2d, wlin, blin.reshape(1, -1))


def _make_scatter(d):
    mesh = plsc.VectorSubcoreMesh(core_axis_name="c", subcore_axis_name="s")
    cp = pltpu.CompilerParams()
    if "needs_layout_passes" in pltpu.CompilerParams.__dataclass_fields__:
        cp = dataclasses.replace(cp, needs_layout_passes=False)
    if d < 128 and "use_tc_tiling_on_sc" in pltpu.CompilerParams.__dataclass_fields__:
        cp = dataclasses.replace(cp, use_tc_tiling_on_sc=False)

    @functools.partial(
        pl.kernel,
        compiler_params=cp,
        out_type=jax.ShapeDtypeStruct((_NC, N_NODES, d), jnp.float32),
        mesh=mesh,
        scratch_types=[
            pltpu.VMEM((_K,), jnp.int32),
            pltpu.VMEM((_K,), jnp.int32),
            pltpu.VMEM((_K,), jnp.float32),
            pltpu.VMEM((_K, d), jnp.float32),
            pltpu.VMEM_SHARED((N_NODES, d), jnp.float32),
            pltpu.SemaphoreType.DMA,
            pltpu.SemaphoreType.DMA,
            pltpu.SemaphoreType.DMA,
        ],
    )
    def sc_kernel(h_hbm, src_hbm, dst_hbm, w_hbm, z_hbm, out_hbm,
                  srcv, dstv, wv, rows, acc, sem, sem_s, sem_dw):
        c = lax.axis_index("c")
        s = lax.axis_index("s")
        wid = s * _NC + c
        r0 = s * _ROWS_PER_TILE

        pltpu.sync_copy(z_hbm.at[pl.ds(r0, _ROWS_PER_TILE)],
                        acc.at[pl.ds(r0, _ROWS_PER_TILE)])

        @pl.when(s == _NS - 1)
        def _():
            pltpu.sync_copy(z_hbm.at[pl.ds(_NS * _ROWS_PER_TILE, _ROWS_REM)],
                            acc.at[pl.ds(_NS * _ROWS_PER_TILE, _ROWS_REM)])

        plsc.subcore_barrier()

        @pl.loop(0, _CHUNKS_PER_W)
        def _(i):
            ci = i * _NW + wid

            @pl.when(ci < _N_CHUNKS)
            def _():
                e0 = ci * _K
                cp_s = pltpu.async_copy(src_hbm.at[pl.ds(e0, _K)], srcv, sem_s)
                cp_d = pltpu.async_copy(dst_hbm.at[pl.ds(e0, _K)], dstv, sem_dw)
                cp_w = pltpu.async_copy(w_hbm.at[pl.ds(e0, _K)], wv, sem_dw)
                cp_s.wait()
                cp_g = pltpu.async_copy(h_hbm.at[srcv], rows, sem)
                cp_d.wait()
                cp_w.wait()
                cp_g.wait()

                @pl.loop(0, _K, step=4)
                def _(k0):
                    for kk in range(4):
                        k = k0 + kk
                        wb = plsc.load_gather(wv, [jnp.full((16,), 0, jnp.int32) + k])
                        for j in range(d // 16):
                            sl = (k, pl.ds(j * 16, 16))
                            rows[sl] = rows[sl] * wb

                pltpu.sync_copy(rows, acc.at[dstv], add=True)

        plsc.subcore_barrier()
        pltpu.sync_copy(acc.at[pl.ds(r0, _ROWS_PER_TILE)],
                        out_hbm.at[c, pl.ds(r0, _ROWS_PER_TILE)])

        @pl.when(s == _NS - 1)
        def _():
            pltpu.sync_copy(acc.at[pl.ds(_NS * _ROWS_PER_TILE, _ROWS_REM)],
                            out_hbm.at[c, pl.ds(_NS * _ROWS_PER_TILE, _ROWS_REM)])

    return sc_kernel


_scatter128 = _make_scatter(128)
_scatter64 = _make_scatter(64)


@jax.jit
def kernel(x, edge_index, batch, edge_weights, W1, b1, W2, b2, W3, b3,
           Wlin, blin):
    src = edge_index[0].astype(jnp.int32)
    dst = edge_index[1].astype(jnp.int32)
    z128 = jnp.zeros((N_NODES, 128), jnp.float32)
    z64 = jnp.zeros((N_NODES, 64), jnp.float32)
    batch2d = batch.astype(jnp.int32).reshape(N_NODES, 1)

    h1 = _mm(x, W1)
    a1 = _scatter128(h1, src, dst, edge_weights, z128)
    h2 = _fuse(a1, b1, W2)
    a2 = _scatter128(h2, src, dst, edge_weights, z128)
    h3 = _fuse(a2, b2, W3)
    a3 = _scatter64(h3, src, dst, edge_weights, z64)
    return _final(a3, b3, batch2d, Wlin, blin)
